# baseline (speedup 1.0000x reference)
import functools

import jax
import jax.numpy as jnp
from jax import lax
from jax.experimental import pallas as pl
from jax.experimental.pallas import tpu as pltpu
from jax.experimental.pallas import tpu_sc as plsc

_NC, _NS, _L = 2, 16, 16
_NW = _NC * _NS
_B = 16384
_D = 32
_V = 1000000
_SR = 128
_NS4 = _V // 4
_BPW = _B // _NW

_TBLK = 4096

_sc_mesh = plsc.VectorSubcoreMesh(core_axis_name="c", subcore_axis_name="s")


@functools.partial(
    pl.kernel,
    out_type=jax.ShapeDtypeStruct((_B, _SR), jnp.float32),
    mesh=_sc_mesh,
    scratch_types=[
        pltpu.VMEM((_BPW,), jnp.int32),
        pltpu.VMEM((_BPW, _SR), jnp.float32),
        pltpu.SemaphoreType.DMA,
    ],
)
def _sc_rowgather(idx_hbm, table_hbm, out_hbm, idx_v, rows_v, sem):
    wid = lax.axis_index("s") * _NC + lax.axis_index("c")
    base = wid * _BPW
    pltpu.sync_copy(idx_hbm.at[pl.ds(base, _BPW)], idx_v)
    pltpu.async_copy(table_hbm.at[idx_v], rows_v, sem).wait()
    pltpu.sync_copy(rows_v, out_hbm.at[pl.ds(base, _BPW)])


def _transpose_body(int_ref, out_ref):
    x = int_ref[...]
    x3 = x.reshape(_D, _TBLK // 4, 4)
    out_ref[...] = jnp.transpose(x3, (1, 2, 0)).reshape(_TBLK // 4, _SR)


def _to_superrows(embT):
    return pl.pallas_call(
        _transpose_body,
        grid=(pl.cdiv(_V, _TBLK),),
        in_specs=[pl.BlockSpec((_D, _TBLK), lambda i: (0, i))],
        out_specs=pl.BlockSpec((_TBLK // 4, _SR), lambda i: (i, 0)),
        out_shape=jax.ShapeDtypeStruct((_NS4, _SR), jnp.float32),
    )(embT)


def _mlp_body(us_ref, ms_ref, ur_ref, mr_ref, w1u_ref, w1m_ref, b1_ref,
              w2_ref, b2_ref, w3_ref, b3_ref, out_ref):
    us = us_ref[...]
    ms = ms_ref[...]
    ur = ur_ref[...]
    mr = mr_ref[...]
    u = jnp.zeros((us.shape[0], _D), jnp.float32)
    m = jnp.zeros((us.shape[0], _D), jnp.float32)
    for r in range(4):
        u = u + jnp.where(ur == r, 1.0, 0.0) * us[:, r * _D:(r + 1) * _D]
        m = m + jnp.where(mr == r, 1.0, 0.0) * ms[:, r * _D:(r + 1) * _D]
    h = jnp.dot(u, w1u_ref[...], preferred_element_type=jnp.float32)
    h = h + jnp.dot(m, w1m_ref[...], preferred_element_type=jnp.float32)
    h = jnp.maximum(h + b1_ref[...], 0.0)
    h = jnp.dot(h, w2_ref[...], preferred_element_type=jnp.float32)
    h = jnp.maximum(h + b2_ref[...], 0.0)
    o = jnp.dot(h, w3_ref[...], preferred_element_type=jnp.float32)
    out_ref[...] = o + b3_ref[...]


def kernel(user, movie, user_emb, movie_emb, W1, b1, W2, b2, W3, b3):
    user = user.astype(jnp.int32)
    movie = movie.astype(jnp.int32)
    ue4 = _to_superrows(user_emb.T)
    me4 = _to_superrows(movie_emb.T)
    u_super = _sc_rowgather(jnp.right_shift(user, 2), ue4)
    m_super = _sc_rowgather(jnp.right_shift(movie, 2), me4)
    blk = 2048
    full = lambda s: pl.BlockSpec(s, lambda i: (0, 0))
    out = pl.pallas_call(
        _mlp_body,
        grid=(_B // blk,),
        in_specs=[
            pl.BlockSpec((blk, _SR), lambda i: (i, 0)),
            pl.BlockSpec((blk, _SR), lambda i: (i, 0)),
            pl.BlockSpec((blk, 1), lambda i: (i, 0)),
            pl.BlockSpec((blk, 1), lambda i: (i, 0)),
            full((_D, 64)), full((_D, 64)), full((1, 64)),
            full((64, 32)), full((1, 32)),
            full((32, 1)), full((1, 1)),
        ],
        out_specs=pl.BlockSpec((blk, 1), lambda i: (i, 0)),
        out_shape=jax.ShapeDtypeStruct((_B, 1), jnp.float32),
    )(u_super, m_super,
      jnp.bitwise_and(user, 3).reshape(_B, 1),
      jnp.bitwise_and(movie, 3).reshape(_B, 1),
      W1[:, :_D].T, W1[:, _D:].T, b1.reshape(1, 64),
      W2.T, b2.reshape(1, 32),
      W3.T, b3.reshape(1, 1))
    return out.reshape(_B)


# speedup vs baseline: 4.5184x; 4.5184x over previous
import functools

import jax
import jax.numpy as jnp
from jax import lax
from jax.experimental import pallas as pl
from jax.experimental.pallas import tpu as pltpu
from jax.experimental.pallas import tpu_sc as plsc

_NC, _NS, _L = 2, 16, 16
_NW = _NC * _NS
_B = 16384
_D = 32
_V = 1000000
_SR = 128
_NS4 = _V // 4
_BPW = _B // _NW

_RBLK = 2000

_sc_mesh = plsc.VectorSubcoreMesh(core_axis_name="c", subcore_axis_name="s")


@functools.partial(
    pl.kernel,
    out_type=jax.ShapeDtypeStruct((_B, _SR), jnp.float32),
    mesh=_sc_mesh,
    scratch_types=[
        pltpu.VMEM((_BPW,), jnp.int32),
        pltpu.VMEM((_BPW, _SR), jnp.float32),
        pltpu.SemaphoreType.DMA,
    ],
)
def _sc_rowgather(idx_hbm, table_hbm, out_hbm, idx_v, rows_v, sem):
    wid = lax.axis_index("s") * _NC + lax.axis_index("c")
    base = wid * _BPW
    pltpu.sync_copy(idx_hbm.at[pl.ds(base, _BPW)], idx_v)
    pltpu.async_copy(table_hbm.at[idx_v], rows_v, sem).wait()
    pltpu.sync_copy(rows_v, out_hbm.at[pl.ds(base, _BPW)])


def _repack_body(a_ref, b_ref, c_ref, d_ref, out_ref):
    out_ref[...] = jnp.concatenate(
        [a_ref[...], b_ref[...], c_ref[...], d_ref[...]], axis=1)


def _to_superrows(emb):
    nb = _NS4 // _RBLK
    mk = lambda k: pl.BlockSpec((_RBLK, _D), lambda i, _k=k: (i + _k * nb, 0))
    return pl.pallas_call(
        _repack_body,
        grid=(nb,),
        in_specs=[mk(0), mk(1), mk(2), mk(3)],
        out_specs=pl.BlockSpec((_RBLK, _SR), lambda i: (i, 0)),
        out_shape=jax.ShapeDtypeStruct((_NS4, _SR), jnp.float32),
    )(emb, emb, emb, emb)


def _mlp_body(us_ref, ms_ref, ur_ref, mr_ref, w1u_ref, w1m_ref, b1_ref,
              w2_ref, b2_ref, w3_ref, b3_ref, out_ref):
    us = us_ref[...]
    ms = ms_ref[...]
    ur = ur_ref[...]
    mr = mr_ref[...]
    u = jnp.zeros((us.shape[0], _D), jnp.float32)
    m = jnp.zeros((us.shape[0], _D), jnp.float32)
    for r in range(4):
        u = u + jnp.where(ur == r, 1.0, 0.0) * us[:, r * _D:(r + 1) * _D]
        m = m + jnp.where(mr == r, 1.0, 0.0) * ms[:, r * _D:(r + 1) * _D]
    h = jnp.dot(u, w1u_ref[...], preferred_element_type=jnp.float32)
    h = h + jnp.dot(m, w1m_ref[...], preferred_element_type=jnp.float32)
    h = jnp.maximum(h + b1_ref[...], 0.0)
    h = jnp.dot(h, w2_ref[...], preferred_element_type=jnp.float32)
    h = jnp.maximum(h + b2_ref[...], 0.0)
    o = jnp.dot(h, w3_ref[...], preferred_element_type=jnp.float32)
    out_ref[...] = o + b3_ref[...]


def kernel(user, movie, user_emb, movie_emb, W1, b1, W2, b2, W3, b3):
    user = user.astype(jnp.int32)
    movie = movie.astype(jnp.int32)
    ue4 = _to_superrows(user_emb)
    me4 = _to_superrows(movie_emb)
    u_super = _sc_rowgather(jnp.mod(user, _NS4), ue4)
    m_super = _sc_rowgather(jnp.mod(movie, _NS4), me4)
    blk = 2048
    full = lambda s: pl.BlockSpec(s, lambda i: (0, 0))
    out = pl.pallas_call(
        _mlp_body,
        grid=(_B // blk,),
        in_specs=[
            pl.BlockSpec((blk, _SR), lambda i: (i, 0)),
            pl.BlockSpec((blk, _SR), lambda i: (i, 0)),
            pl.BlockSpec((blk, 1), lambda i: (i, 0)),
            pl.BlockSpec((blk, 1), lambda i: (i, 0)),
            full((_D, 64)), full((_D, 64)), full((1, 64)),
            full((64, 32)), full((1, 32)),
            full((32, 1)), full((1, 1)),
        ],
        out_specs=pl.BlockSpec((blk, 1), lambda i: (i, 0)),
        out_shape=jax.ShapeDtypeStruct((_B, 1), jnp.float32),
    )(u_super, m_super,
      jnp.floor_divide(user, _NS4).reshape(_B, 1),
      jnp.floor_divide(movie, _NS4).reshape(_B, 1),
      W1[:, :_D].T, W1[:, _D:].T, b1.reshape(1, 64),
      W2.T, b2.reshape(1, 32),
      W3.T, b3.reshape(1, 1))
    return out.reshape(_B)


# speedup vs baseline: 5.2486x; 1.1616x over previous
import functools

import jax
import jax.numpy as jnp
from jax import lax
from jax.experimental import pallas as pl
from jax.experimental.pallas import tpu as pltpu
from jax.experimental.pallas import tpu_sc as plsc

_NC, _NS, _L = 2, 16, 16
_NW = _NC * _NS
_B = 16384
_D = 32
_V = 1000000
_SR = 128
_NS4 = _V // 4
_BPW = _B // _NW

_RBLK = 2000

_sc_mesh = plsc.VectorSubcoreMesh(core_axis_name="c", subcore_axis_name="s")


@functools.partial(
    pl.kernel,
    out_type=jax.ShapeDtypeStruct((_B, _SR), jnp.float32),
    mesh=_sc_mesh,
    scratch_types=[
        pltpu.VMEM((_BPW,), jnp.int32),
        pltpu.VMEM((_BPW, _SR), jnp.float32),
        pltpu.SemaphoreType.DMA,
    ],
)
def _sc_rowgather(idx_hbm, table_hbm, out_hbm, idx_v, rows_v, sem):
    wid = lax.axis_index("s") * _NC + lax.axis_index("c")
    base = wid * _BPW
    pltpu.sync_copy(idx_hbm.at[pl.ds(base, _BPW)], idx_v)
    pltpu.async_copy(table_hbm.at[idx_v], rows_v, sem).wait()
    pltpu.sync_copy(rows_v, out_hbm.at[pl.ds(base, _BPW)])


def _repack_body(a_ref, b_ref, c_ref, d_ref, out_ref):
    out_ref[...] = jnp.concatenate(
        [a_ref[...], b_ref[...], c_ref[...], d_ref[...]], axis=1)


def _to_superrows(emb):
    nb = _NS4 // _RBLK
    mk = lambda k: pl.BlockSpec((_RBLK, _D), lambda i, _k=k: (i + _k * nb, 0))
    return pl.pallas_call(
        _repack_body,
        grid=(nb,),
        in_specs=[mk(0), mk(1), mk(2), mk(3)],
        out_specs=pl.BlockSpec((_RBLK, _SR), lambda i: (i, 0)),
        out_shape=jax.ShapeDtypeStruct((_NS4, _SR), jnp.float32),
    )(emb, emb, emb, emb)


def _mlp_body(us_ref, ms_ref, ur_ref, mr_ref, w1u_ref, w1m_ref, b1_ref,
              w2_ref, b2_ref, w3_ref, b3_ref, out_ref):
    us = us_ref[...]
    ms = ms_ref[...]
    ur = ur_ref[...]
    mr = mr_ref[...]
    u = jnp.zeros((us.shape[0], _D), jnp.float32)
    m = jnp.zeros((us.shape[0], _D), jnp.float32)
    for r in range(4):
        u = u + jnp.where(ur == r, 1.0, 0.0) * us[:, r * _D:(r + 1) * _D]
        m = m + jnp.where(mr == r, 1.0, 0.0) * ms[:, r * _D:(r + 1) * _D]
    h = jnp.dot(u, w1u_ref[...], preferred_element_type=jnp.float32)
    h = h + jnp.dot(m, w1m_ref[...], preferred_element_type=jnp.float32)
    h = jnp.maximum(h + b1_ref[...], 0.0)
    h = jnp.dot(h, w2_ref[...], preferred_element_type=jnp.float32)
    h = jnp.maximum(h + b2_ref[...], 0.0)
    o = jnp.dot(h, w3_ref[...], preferred_element_type=jnp.float32)
    out_ref[...] = o + b3_ref[...]


def kernel(user, movie, user_emb, movie_emb, W1, b1, W2, b2, W3, b3):
    user = user.astype(jnp.int32)
    movie = movie.astype(jnp.int32)
    ue4 = user_emb.reshape(_NS4, _SR)
    me4 = movie_emb.reshape(_NS4, _SR)
    u_super = _sc_rowgather(jnp.right_shift(user, 2), ue4)
    m_super = _sc_rowgather(jnp.right_shift(movie, 2), me4)
    blk = 2048
    full = lambda s: pl.BlockSpec(s, lambda i: (0, 0))
    out = pl.pallas_call(
        _mlp_body,
        grid=(_B // blk,),
        in_specs=[
            pl.BlockSpec((blk, _SR), lambda i: (i, 0)),
            pl.BlockSpec((blk, _SR), lambda i: (i, 0)),
            pl.BlockSpec((blk, 1), lambda i: (i, 0)),
            pl.BlockSpec((blk, 1), lambda i: (i, 0)),
            full((_D, 64)), full((_D, 64)), full((1, 64)),
            full((64, 32)), full((1, 32)),
            full((32, 1)), full((1, 1)),
        ],
        out_specs=pl.BlockSpec((blk, 1), lambda i: (i, 0)),
        out_shape=jax.ShapeDtypeStruct((_B, 1), jnp.float32),
    )(u_super, m_super,
      jnp.bitwise_and(user, 3).reshape(_B, 1),
      jnp.bitwise_and(movie, 3).reshape(_B, 1),
      W1[:, :_D].T, W1[:, _D:].T, b1.reshape(1, 64),
      W2.T, b2.reshape(1, 32),
      W3.T, b3.reshape(1, 1))
    return out.reshape(_B)


# speedup vs baseline: 5.4749x; 1.0431x over previous
import functools

import jax
import jax.numpy as jnp
from jax import lax
from jax.experimental import pallas as pl
from jax.experimental.pallas import tpu as pltpu
from jax.experimental.pallas import tpu_sc as plsc

_NC, _NS, _L = 2, 16, 16
_NW = _NC * _NS
_B = 16384
_D = 32
_V = 1000000
_SR = 128
_NS4 = _V // 4
_BPW = _B // _NW

_RBLK = 1000

_sc_mesh = plsc.VectorSubcoreMesh(core_axis_name="c", subcore_axis_name="s")


@functools.partial(
    pl.kernel,
    out_type=jax.ShapeDtypeStruct((_B, _SR), jnp.float32),
    mesh=_sc_mesh,
    scratch_types=[
        pltpu.VMEM((_BPW,), jnp.int32),
        pltpu.VMEM((_BPW, _SR), jnp.float32),
        pltpu.SemaphoreType.DMA,
    ],
)
def _sc_rowgather(idx_hbm, table_hbm, out_hbm, idx_v, rows_v, sem):
    wid = lax.axis_index("s") * _NC + lax.axis_index("c")
    base = wid * _BPW
    pltpu.sync_copy(idx_hbm.at[pl.ds(base, _BPW)], idx_v)
    pltpu.async_copy(table_hbm.at[idx_v], rows_v, sem).wait()
    pltpu.sync_copy(rows_v, out_hbm.at[pl.ds(base, _BPW)])


def _repack_body(in_ref, out_ref):
    x = in_ref[...]
    out_ref[0] = jnp.concatenate([x[:, r, :] for r in range(4)], axis=1)
    out_ref[1] = jnp.concatenate([x[:, r, :] for r in range(4, 8)], axis=1)


def _to_superrows(emb):
    nt = _V // 8
    return pl.pallas_call(
        _repack_body,
        grid=(nt // _RBLK,),
        in_specs=[pl.BlockSpec((_RBLK, 8, _D), lambda i: (i, 0, 0))],
        out_specs=pl.BlockSpec((2, _RBLK, _SR), lambda i: (0, i, 0)),
        out_shape=jax.ShapeDtypeStruct((2, nt, _SR), jnp.float32),
    )(emb.reshape(nt, 8, _D)).reshape(_NS4, _SR)


def _mlp_body(us_ref, ms_ref, ur_ref, mr_ref, w1u_ref, w1m_ref, b1_ref,
              w2_ref, b2_ref, w3_ref, b3_ref, out_ref):
    us = us_ref[...]
    ms = ms_ref[...]
    ur = ur_ref[...]
    mr = mr_ref[...]
    u = jnp.zeros((us.shape[0], _D), jnp.float32)
    m = jnp.zeros((us.shape[0], _D), jnp.float32)
    for r in range(4):
        u = u + jnp.where(ur == r, 1.0, 0.0) * us[:, r * _D:(r + 1) * _D]
        m = m + jnp.where(mr == r, 1.0, 0.0) * ms[:, r * _D:(r + 1) * _D]
    h = jnp.dot(u, w1u_ref[...], preferred_element_type=jnp.float32)
    h = h + jnp.dot(m, w1m_ref[...], preferred_element_type=jnp.float32)
    h = jnp.maximum(h + b1_ref[...], 0.0)
    h = jnp.dot(h, w2_ref[...], preferred_element_type=jnp.float32)
    h = jnp.maximum(h + b2_ref[...], 0.0)
    o = jnp.dot(h, w3_ref[...], preferred_element_type=jnp.float32)
    out_ref[...] = o + b3_ref[...]


def kernel(user, movie, user_emb, movie_emb, W1, b1, W2, b2, W3, b3):
    user = user.astype(jnp.int32)
    movie = movie.astype(jnp.int32)
    ue4 = _to_superrows(user_emb)
    me4 = _to_superrows(movie_emb)
    sidx = lambda i: (jnp.bitwise_and(jnp.right_shift(i, 2), 1) * (_V // 8)
                      + jnp.right_shift(i, 3))
    u_super = _sc_rowgather(sidx(user), ue4)
    m_super = _sc_rowgather(sidx(movie), me4)
    blk = 2048
    full = lambda s: pl.BlockSpec(s, lambda i: (0, 0))
    out = pl.pallas_call(
        _mlp_body,
        grid=(_B // blk,),
        in_specs=[
            pl.BlockSpec((blk, _SR), lambda i: (i, 0)),
            pl.BlockSpec((blk, _SR), lambda i: (i, 0)),
            pl.BlockSpec((blk, 1), lambda i: (i, 0)),
            pl.BlockSpec((blk, 1), lambda i: (i, 0)),
            full((_D, 64)), full((_D, 64)), full((1, 64)),
            full((64, 32)), full((1, 32)),
            full((32, 1)), full((1, 1)),
        ],
        out_specs=pl.BlockSpec((blk, 1), lambda i: (i, 0)),
        out_shape=jax.ShapeDtypeStruct((_B, 1), jnp.float32),
    )(u_super, m_super,
      jnp.bitwise_and(user, 3).reshape(_B, 1),
      jnp.bitwise_and(movie, 3).reshape(_B, 1),
      W1[:, :_D].T, W1[:, _D:].T, b1.reshape(1, 64),
      W2.T, b2.reshape(1, 32),
      W3.T, b3.reshape(1, 1))
    return out.reshape(_B)
